# Initial kernel scaffold; baseline (speedup 1.0000x reference)
import jax
import jax.numpy as jnp
from jax.experimental import pallas as pl


def kernel(norm_X, norm_E, HHT_idx, HHT_val, H_idx, H_val, HT_idx, HT_val, HTH_idx, HTH_val, W0, W1):
    raise NotImplementedError("write your pallas kernel here")


# speedup vs baseline: 2.9378x; 2.9378x over previous
import functools

import jax
import jax.numpy as jnp
from jax import lax
from jax.experimental import pallas as pl
from jax.experimental.pallas import tpu as pltpu
from jax.experimental.pallas import tpu_sc as plsc

NEG = 0.2
NC = 2
NS = 16
L = 16
K = 128


def _lrelu(v):
    return jnp.maximum(v, NEG * v)


def _mm2_body(x_ref, e_ref, w_ref, ox_ref, oe_ref):
    ox_ref[...] = jnp.dot(x_ref[...], w_ref[...],
                          preferred_element_type=jnp.float32)
    oe_ref[...] = jnp.dot(e_ref[...], w_ref[...],
                          preferred_element_type=jnp.float32)


def _mm2(x, e, w):
    return pl.pallas_call(
        _mm2_body,
        out_shape=[
            jax.ShapeDtypeStruct((x.shape[0], w.shape[1]), jnp.float32),
            jax.ShapeDtypeStruct((e.shape[0], w.shape[1]), jnp.float32),
        ],
    )(x, e, w)


def _mid_body(a_ref, c_ref, b_ref, d_ref, w_ref, ox_ref, oy_ref):
    sx = _lrelu(a_ref[...]) + _lrelu(c_ref[...])
    ox_ref[...] = jnp.dot(sx, w_ref[...], preferred_element_type=jnp.float32)
    sy = _lrelu(d_ref[...]) + _lrelu(b_ref[...])
    oy_ref[...] = jnp.dot(sy, w_ref[...], preferred_element_type=jnp.float32)


def _mid(a, c, b, d, w):
    return pl.pallas_call(
        _mid_body,
        out_shape=[
            jax.ShapeDtypeStruct((a.shape[0], w.shape[1]), jnp.float32),
            jax.ShapeDtypeStruct((b.shape[0], w.shape[1]), jnp.float32),
        ],
    )(a, c, b, d, w)


def _fin_body(a_ref, c_ref, b_ref, d_ref, ox_ref, oy_ref):
    ox_ref[...] = _lrelu(a_ref[...]) + _lrelu(c_ref[...])
    oy_ref[...] = _lrelu(d_ref[...]) + _lrelu(b_ref[...])


def _fin(a, c, b, d):
    return pl.pallas_call(
        _fin_body,
        out_shape=[
            jax.ShapeDtypeStruct(a.shape, jnp.float32),
            jax.ShapeDtypeStruct(b.shape, jnp.float32),
        ],
    )(a, c, b, d)


@functools.lru_cache(maxsize=None)
def _sc_spmm_kernel(w, n_big, n_small, nch_hht, nch_hth, nch_h, nch_ht):
    mesh = plsc.VectorSubcoreMesh(core_axis_name="c", subcore_axis_name="s",
                                  num_cores=NC, num_subcores=NS)
    out_type = [
        jax.ShapeDtypeStruct((n_big, w), jnp.float32),
        jax.ShapeDtypeStruct((n_big, w), jnp.float32),
        jax.ShapeDtypeStruct((n_small, w), jnp.float32),
        jax.ShapeDtypeStruct((n_small, w), jnp.float32),
    ]
    scratch = [
        pltpu.VMEM_SHARED((n_big, w), jnp.float32),
        pltpu.VMEM((K,), jnp.int32),
        pltpu.VMEM((K,), jnp.int32),
        pltpu.VMEM((K,), jnp.float32),
        pltpu.VMEM((K, w), jnp.float32),
        pltpu.SemaphoreType.DMA,
    ]

    def body(sx, sy,
             r_hht, c_hht, v_hht, r_h, c_h, v_h,
             r_ht, c_ht, v_ht, r_hth, c_hth, v_hth,
             o_hht, o_h, o_ht, o_hth,
             acc, ridx, cidx, vbuf, buf, sem):
        core = lax.axis_index("c")
        tile = lax.axis_index("s")

        def _edges(i, carry):
            iv = jnp.full((L,), i, dtype=jnp.int32)
            vv = plsc.load_gather(vbuf, [iv])
            for j in range(w // L):
                sl = pl.ds(j * L, L)
                buf[i, sl] = buf[i, sl] * vv
            return carry

        def _phase(row_h, col_h, val_h, src_h, out_h, nch, nrows):
            def _zrow(r, carry):
                for j in range(w // L):
                    buf[r, pl.ds(j * L, L)] = jnp.zeros((L,), jnp.float32)
                return carry
            lax.fori_loop(0, K, _zrow, 0)
            nz = nrows // K
            for j in range(-(-nz // NS)):
                kk = tile + NS * j
                @pl.when(kk < nz)
                def _():
                    pltpu.sync_copy(buf, acc.at[pl.ds(kk * K, K)])
            plsc.subcore_barrier()

            base = tile * (nch * K)

            def chunk(ci, carry):
                off = base + ci * K
                pltpu.sync_copy(row_h.at[pl.ds(off, K)], ridx)
                pltpu.sync_copy(col_h.at[pl.ds(off, K)], cidx)
                pltpu.sync_copy(val_h.at[pl.ds(off, K)], vbuf)
                pltpu.async_copy(src_h.at[cidx], buf, sem).wait()
                lax.fori_loop(0, K, _edges, 0)
                pltpu.sync_copy(buf, acc.at[ridx], add=True)
                return carry
            lax.fori_loop(0, nch, chunk, 0)
            plsc.subcore_barrier()

            for j in range(-(-nz // NS)):
                kk = tile + NS * j
                @pl.when(kk < nz)
                def _():
                    pltpu.sync_copy(acc.at[pl.ds(kk * K, K)],
                                    out_h.at[pl.ds(kk * K, K)])
            plsc.subcore_barrier()

        @pl.when(core == 0)
        def _():
            _phase(r_hht, c_hht, v_hht, sx, o_hht, nch_hht, n_big)
            _phase(r_hth, c_hth, v_hth, sy, o_hth, nch_hth, n_small)

        @pl.when(core == 1)
        def _():
            _phase(r_h, c_h, v_h, sy, o_h, nch_h, n_big)
            _phase(r_ht, c_ht, v_ht, sx, o_ht, nch_ht, n_small)

    return pl.kernel(
        body, out_type=out_type, mesh=mesh, scratch_types=scratch,
        compiler_params=pltpu.CompilerParams(needs_layout_passes=False,
                                             use_tc_tiling_on_sc=False))


def _pad_edges(idx, val, n_src, n_dst):
    e = idx.shape[1]
    target = -(-e // (NS * K)) * (NS * K)
    pad = target - e
    ar = jnp.arange(pad, dtype=jnp.int32)
    rows = jnp.concatenate([idx[0], ar % n_dst])
    cols = jnp.concatenate([idx[1], ar % n_src])
    vals = jnp.concatenate([val, jnp.zeros((pad,), jnp.float32)])
    return rows, cols, vals, target // (NS * K)


def kernel(norm_X, norm_E, HHT_idx, HHT_val, H_idx, H_val,
           HT_idx, HT_val, HTH_idx, HTH_val, W0, W1):
    n = norm_X.shape[0]
    m = norm_E.shape[0]
    hid = W0.shape[1]
    out_w = W1.shape[1]
    n_pad = -(-n // K) * K
    m_pad = -(-m // K) * K

    r1, c1, v1, nch1 = _pad_edges(HHT_idx, HHT_val, n, n)
    r2, c2, v2, nch2 = _pad_edges(H_idx, H_val, m, n)
    r3, c3, v3, nch3 = _pad_edges(HT_idx, HT_val, n, m)
    r4, c4, v4, nch4 = _pad_edges(HTH_idx, HTH_val, m, m)

    sx0, sy0 = _mm2(norm_X, norm_E, W0)
    k0 = _sc_spmm_kernel(hid, n_pad, m_pad, nch1, nch4, nch2, nch3)
    a_hht, a_h, a_ht, a_hth = k0(sx0, sy0, r1, c1, v1, r2, c2, v2,
                                 r3, c3, v3, r4, c4, v4)
    sx1, sy1 = _mid(a_hht, a_h, a_hth, a_ht, W1)
    k1 = _sc_spmm_kernel(out_w, n_pad, m_pad, nch1, nch4, nch2, nch3)
    b_hht, b_h, b_ht, b_hth = k1(sx1, sy1, r1, c1, v1, r2, c2, v2,
                                 r3, c3, v3, r4, c4, v4)
    x, y = _fin(b_hht, b_h, b_hth, b_ht)
    return x[:n], y[:m]
